# baseline (speedup 1.0000x reference)
import functools

import jax
import jax.numpy as jnp
from jax import lax
from jax.experimental import pallas as pl
from jax.experimental.pallas import tpu as pltpu
from jax.experimental.pallas import tpu_sc as plsc

NC = 2
NS = 16
NW = NC * NS
L = 16


@functools.partial(jax.jit, static_argnames=("chunk_rows",))
def _sc_lookup_sum(zt, w_pad, chunk_rows):
    n_rows, n_cols = zt.shape
    cols_per_worker = n_cols // NW
    n_chunks = n_rows // chunk_rows
    vecs_per_row = cols_per_worker // L
    mesh = plsc.VectorSubcoreMesh(
        core_axis_name="c", subcore_axis_name="s", num_cores=NC, num_subcores=NS
    )

    @functools.partial(
        pl.kernel,
        out_type=jax.ShapeDtypeStruct((NW * L,), jnp.float32),
        mesh=mesh,
        compiler_params=pltpu.CompilerParams(
            needs_layout_passes=False,
            use_tc_tiling_on_sc=True,
            skip_device_barrier=True,
        ),
        scratch_types=[
            pltpu.VMEM((128 * 128,), jnp.float32),
            pltpu.VMEM((chunk_rows, cols_per_worker), jnp.int32),
            pltpu.VMEM((chunk_rows, cols_per_worker), jnp.int32),
            pltpu.VMEM((L,), jnp.float32),
            pltpu.SemaphoreType.DMA,
            pltpu.SemaphoreType.DMA,
        ],
    )
    def k(z_hbm, w_hbm, out_hbm, tbl_v, buf0, buf1, acc_v, sem0, sem1):
        wid = lax.axis_index("s") * NC + lax.axis_index("c")
        col0 = wid * cols_per_worker

        bufs = (buf0, buf1)
        sems = (sem0, sem1)
        copies = [
            pltpu.async_copy(
                z_hbm.at[
                    pl.ds(c * chunk_rows, chunk_rows),
                    pl.ds(col0, cols_per_worker),
                ],
                bufs[c % 2],
                sems[c % 2],
            )
            for c in range(min(2, n_chunks))
        ]

        pltpu.sync_copy(w_hbm, tbl_v)

        zero = jnp.zeros((L,), jnp.float32)
        accs = (zero,) * 4
        for c in range(n_chunks):
            buf = bufs[c % 2]
            copies[c].wait()

            def body(r, a, buf=buf):
                a = list(a)
                for j in range(vecs_per_row // 2):
                    ia = buf[r, pl.ds(2 * j * L, L)]
                    ib = buf[r, pl.ds((2 * j + 1) * L, L)]
                    idx = ia * 128 + ib
                    a[j % 4] = a[j % 4] + plsc.load_gather(tbl_v, [idx])
                return tuple(a)

            accs = plsc.parallel_loop(0, chunk_rows, 1, unroll=2, carry=accs)(body)

            nxt = c + 2
            if nxt < n_chunks:
                copies.append(
                    pltpu.async_copy(
                        z_hbm.at[
                            pl.ds(nxt * chunk_rows, chunk_rows),
                            pl.ds(col0, cols_per_worker),
                        ],
                        bufs[nxt % 2],
                        sems[nxt % 2],
                    )
                )

        acc_v[...] = (accs[0] + accs[1]) + (accs[2] + accs[3])
        pltpu.sync_copy(acc_v, out_hbm.at[pl.ds(wid * L, L)])

    return k(zt, w_pad)


def kernel(Z, ref_weight):
    w_pad = jnp.zeros((128,), jnp.float32).at[: ref_weight.shape[0]].set(
        ref_weight.reshape(-1)
    )
    w_pair = (w_pad[:, None] + w_pad[None, :]).reshape(128 * 128)
    partials = _sc_lookup_sum(Z.T, w_pair, 40)
    return jnp.sum(partials)


# speedup vs baseline: 1.0541x; 1.0541x over previous
import functools

import jax
import jax.numpy as jnp
from jax import lax
from jax.experimental import pallas as pl
from jax.experimental.pallas import tpu as pltpu
from jax.experimental.pallas import tpu_sc as plsc

NC = 2
NS = 16
NW = NC * NS
L = 16


@functools.partial(jax.jit, static_argnames=("chunk_rows",))
def _sc_lookup_sum(zt, w_pad, chunk_rows):
    n_rows, n_cols = zt.shape
    cols_per_worker = n_cols // NW
    n_chunks = n_rows // chunk_rows
    vecs_per_row = cols_per_worker // L
    mesh = plsc.VectorSubcoreMesh(
        core_axis_name="c", subcore_axis_name="s", num_cores=NC, num_subcores=NS
    )

    @functools.partial(
        pl.kernel,
        out_type=jax.ShapeDtypeStruct((NW * L,), jnp.float32),
        mesh=mesh,
        compiler_params=pltpu.CompilerParams(
            needs_layout_passes=False,
            use_tc_tiling_on_sc=True,
            skip_device_barrier=True,
        ),
        scratch_types=[
            pltpu.VMEM((128 * 128,), jnp.float32),
            pltpu.VMEM((chunk_rows, cols_per_worker), jnp.int32),
            pltpu.VMEM((chunk_rows, cols_per_worker), jnp.int32),
            pltpu.VMEM((L,), jnp.float32),
            pltpu.SemaphoreType.DMA,
            pltpu.SemaphoreType.DMA,
        ],
    )
    def k(z_hbm, w_hbm, out_hbm, tbl_v, buf0, buf1, acc_v, sem0, sem1):
        wid = lax.axis_index("s") * NC + lax.axis_index("c")
        col0 = wid * cols_per_worker

        bufs = (buf0, buf1)
        sems = (sem0, sem1)
        copies = [
            pltpu.async_copy(
                z_hbm.at[
                    pl.ds(c * chunk_rows, chunk_rows),
                    pl.ds(col0, cols_per_worker),
                ],
                bufs[c % 2],
                sems[c % 2],
            )
            for c in range(min(2, n_chunks))
        ]

        pltpu.sync_copy(w_hbm, tbl_v)

        half = cols_per_worker // 2
        pairs = half // (2 * L)

        def load_half(buf, r, c0):
            out = []
            for k in range(pairs):
                ia = buf[r, pl.ds(c0 + 2 * k * L, L)]
                ib = buf[r, pl.ds(c0 + (2 * k + 1) * L, L)]
                out.append(ia * 128 + ib)
            return tuple(out)

        zero = jnp.zeros((L,), jnp.float32)
        accs = (zero,) * 4
        n_halves = 2 * chunk_rows
        for c in range(n_chunks):
            buf = bufs[c % 2]
            copies[c].wait()

            idx8 = load_half(buf, 0, 0)

            def body(h, carry, buf=buf):
                idx8, a = carry
                a = list(a)
                for k in range(pairs):
                    a[k % 4] = a[k % 4] + plsc.load_gather(tbl_v, [idx8[k]])
                hn = h + 1
                r = lax.shift_right_logical(hn, 1)
                c0 = lax.shift_left(jnp.bitwise_and(hn, 1), 8)
                return (load_half(buf, r, c0), tuple(a))

            idx8, accs = lax.fori_loop(0, n_halves - 1, body, (idx8, accs))
            accs = list(accs)
            for k in range(pairs):
                accs[k % 4] = accs[k % 4] + plsc.load_gather(tbl_v, [idx8[k]])
            accs = tuple(accs)

            nxt = c + 2
            if nxt < n_chunks:
                copies.append(
                    pltpu.async_copy(
                        z_hbm.at[
                            pl.ds(nxt * chunk_rows, chunk_rows),
                            pl.ds(col0, cols_per_worker),
                        ],
                        bufs[nxt % 2],
                        sems[nxt % 2],
                    )
                )

        acc_v[...] = (accs[0] + accs[1]) + (accs[2] + accs[3])
        pltpu.sync_copy(acc_v, out_hbm.at[pl.ds(wid * L, L)])

    return k(zt, w_pad)


def kernel(Z, ref_weight):
    w_pad = jnp.zeros((128,), jnp.float32).at[: ref_weight.shape[0]].set(
        ref_weight.reshape(-1)
    )
    w_pair = (w_pad[:, None] + w_pad[None, :]).reshape(128 * 128)
    partials = _sc_lookup_sum(Z.T, w_pair, 40)
    return jnp.sum(partials)
